# baseline (speedup 1.0000x reference)
import functools

import jax
import jax.numpy as jnp
import numpy as np
from jax import lax
from jax.experimental import pallas as pl
from jax.experimental.pallas import tpu as pltpu
from jax.experimental.pallas import tpu_sc as plsc

NUM_BINS = 256
H = 512
W = 512
NUM_IMGS = 4
HIST = NUM_BINS * NUM_BINS
TOTAL_BINS = NUM_IMGS * HIST
NTILES = 32
PER_TILE = (NUM_IMGS * H * W) // NTILES
CHUNK = 128
NCHUNK = PER_TILE // CHUNK
CORE_BINS = 2 * HIST
ZCHUNK = CORE_BINS // 16


def _grad_index_body(x0_ref, x1_ref, x2_ref, x3_ref, o_ref):
    img = pl.program_id(0)

    def one(x_ref):
        x = x_ref[0].astype(jnp.bfloat16).astype(jnp.float32)
        xd = jnp.pad(lax.slice(x, (0, 0), (H - 1, W)), ((1, 0), (0, 0)))
        xu = jnp.pad(lax.slice(x, (1, 0), (H, W)), ((0, 1), (0, 0)))
        vsm = (xd + 2.0 * x) + xu
        vdf = xu - xd

        def left(a):
            return jnp.pad(lax.slice(a, (0, 1), (H, W)), ((0, 0), (0, 1)))

        def right(a):
            return jnp.pad(lax.slice(a, (0, 0), (H, W - 1)), ((0, 0), (1, 0)))

        gx = left(vsm) - right(vsm)
        gy = (right(vdf) + 2.0 * vdf) + left(vdf)
        g = jnp.sqrt(gx * gx + gy * gy)
        gi = jnp.mod(jnp.floor(g * 255.0).astype(jnp.int32), NUM_BINS)
        q = jnp.pad(lax.slice(gi, (0, 1), (H, W)), ((0, 0), (0, 1)))
        col = lax.broadcasted_iota(jnp.int32, (H, W), 1)
        k = jnp.where(col < W - 1, gi * NUM_BINS + q, 0)
        o_ref[...] = jnp.reshape(k, (8, PER_TILE))

    for n, r in enumerate((x0_ref, x1_ref, x2_ref, x3_ref)):
        @pl.when(img == n)
        def _(r=r):
            one(r)


def _grad_index(x0, x1, x2, x3):
    spec = pl.BlockSpec((1, H, W), lambda i: (0, 0, 0))
    return pl.pallas_call(
        _grad_index_body,
        grid=(NUM_IMGS,),
        in_specs=[spec, spec, spec, spec],
        out_specs=pl.BlockSpec((8, PER_TILE), lambda i: (i, 0)),
        out_shape=jax.ShapeDtypeStruct((NTILES, PER_TILE), jnp.int32),
    )(x0, x1, x2, x3)


@functools.cache
def _sc_hist_fn():
    mesh = plsc.VectorSubcoreMesh(core_axis_name="c", subcore_axis_name="s")

    @functools.partial(
        pl.kernel,
        mesh=mesh,
        out_type=jax.ShapeDtypeStruct((NTILES, HIST), jnp.float32),
        compiler_params=pltpu.CompilerParams(needs_layout_passes=False),
        scratch_types=[
            pltpu.VMEM((PER_TILE,), jnp.int32),
            pltpu.VMEM((HIST,), jnp.float32),
        ],
    )
    def _sc_hist(idx_hbm, zeros_hbm, out_hbm, idx_v, hist_v):
        c = lax.axis_index("c")
        s = lax.axis_index("s")
        wid = c * 16 + s
        pltpu.sync_copy(zeros_hbm, hist_v)
        pltpu.sync_copy(idx_hbm.at[wid], idx_v)
        ones16 = jnp.ones((16,), jnp.float32)
        unroll = 16

        def body(j, carry):
            for b in range(unroll):
                iv = idx_v[pl.ds(j * (16 * unroll) + b * 16, 16)]
                plsc.addupdate_scatter(hist_v, [iv], ones16)
            return carry

        lax.fori_loop(0, PER_TILE // (16 * unroll), body, 0)
        pltpu.sync_copy(hist_v, out_hbm.at[wid])

    return _sc_hist


def _features_body(h_ref, o_ref):
    def img_hist(img):
        part = lax.slice(h_ref[...], (img * 8, 0), (img * 8 + 8, HIST))
        return jnp.reshape(jnp.sum(part, axis=0), (NUM_BINS, NUM_BINS))
    iif = lax.broadcasted_iota(jnp.int32, (NUM_BINS, NUM_BINS), 0).astype(jnp.float32)
    jjf = lax.broadcasted_iota(jnp.int32, (NUM_BINS, NUM_BINS), 1).astype(jnp.float32)
    corner = jnp.where((iif == 0.0) & (jjf == 0.0), float(H), 0.0)

    arf = lax.broadcasted_iota(jnp.int32, (1, NUM_BINS), 1).astype(jnp.float32)

    def feats(hc):
        g = hc + hc.T
        g = g / jnp.sum(g)
        energy = jnp.sum(g * g)
        rs = jnp.sum(g, axis=1).reshape(1, NUM_BINS)
        cs = jnp.sum(g, axis=0).reshape(1, NUM_BINS)
        mean_i = jnp.sum(arf * rs)
        mean_j = jnp.sum(arf * cs)
        std_i = jnp.sqrt(jnp.sum((mean_i - arf) ** 2 * cs))
        std_j = jnp.sqrt(jnp.sum((mean_j - arf) ** 2 * cs))
        b = (mean_j - arf) / (std_i * std_j)
        t = jnp.sum(g * b, axis=1).reshape(1, NUM_BINS)
        corr = jnp.sum((mean_i - arf) * t)
        corr = jnp.where((std_i == 0.0) | (std_j == 0.0), 0.0, corr)
        entropy = -jnp.sum(g * jnp.log(g + 1e-08))
        return energy, corr, entropy

    f = [feats(img_hist(i) - corner) for i in range(NUM_IMGS)]
    total = (jnp.abs(f[0][0] - f[1][0]) + jnp.abs(f[0][1] - f[1][1])
             + jnp.abs(f[0][2] - f[1][2]) + jnp.abs(f[2][0] - f[3][0])
             + jnp.abs(f[2][1] - f[3][1]) + jnp.abs(f[2][2] - f[3][2]))
    o_ref[0, 0] = total


def _features(hists):
    return pl.pallas_call(
        _features_body,
        out_specs=pl.BlockSpec(memory_space=pltpu.SMEM),
        out_shape=jax.ShapeDtypeStruct((1, 1), jnp.float32),
    )(hists)


def kernel(real_image, synthesized_image, real_outline, synthesized_outline):
    idx = _grad_index(real_image, synthesized_image, real_outline,
                      synthesized_outline)
    zeros = np.zeros((HIST,), np.float32)
    hist = _sc_hist_fn()(idx, zeros)
    total = _features(hist)
    return total.reshape(())


# speedup vs baseline: 1.1515x; 1.1515x over previous
import functools

import jax
import jax.numpy as jnp
import numpy as np
from jax import lax
from jax.experimental import pallas as pl
from jax.experimental.pallas import tpu as pltpu
from jax.experimental.pallas import tpu_sc as plsc

NUM_BINS = 256
H = 512
W = 512
NUM_IMGS = 4
HIST = NUM_BINS * NUM_BINS
TOTAL_BINS = NUM_IMGS * HIST
NTILES = 32
PER_TILE = (NUM_IMGS * H * W) // NTILES
CHUNK = 128
NCHUNK = PER_TILE // CHUNK
CORE_BINS = 2 * HIST
ZCHUNK = CORE_BINS // 16


def _grad_index_body(x0_ref, x1_ref, x2_ref, x3_ref, o_ref):
    img = pl.program_id(0)

    def one(x_ref):
        x = x_ref[0].astype(jnp.bfloat16).astype(jnp.float32)
        xd = jnp.pad(lax.slice(x, (0, 0), (H - 1, W)), ((1, 0), (0, 0)))
        xu = jnp.pad(lax.slice(x, (1, 0), (H, W)), ((0, 1), (0, 0)))
        vsm = (xd + 2.0 * x) + xu
        vdf = xu - xd

        def left(a):
            return jnp.pad(lax.slice(a, (0, 1), (H, W)), ((0, 0), (0, 1)))

        def right(a):
            return jnp.pad(lax.slice(a, (0, 0), (H, W - 1)), ((0, 0), (1, 0)))

        gx = left(vsm) - right(vsm)
        gy = (right(vdf) + 2.0 * vdf) + left(vdf)
        g = jnp.sqrt(gx * gx + gy * gy)
        gi = jnp.mod(jnp.floor(g * 255.0).astype(jnp.int32), NUM_BINS)
        q = jnp.pad(lax.slice(gi, (0, 1), (H, W)), ((0, 0), (0, 1)))
        col = lax.broadcasted_iota(jnp.int32, (H, W), 1)
        k = jnp.where(col < W - 1, gi * NUM_BINS + q, 0)
        o_ref[...] = jnp.reshape(k, (8, PER_TILE))

    for n, r in enumerate((x0_ref, x1_ref, x2_ref, x3_ref)):
        @pl.when(img == n)
        def _(r=r):
            one(r)


def _grad_index(x0, x1, x2, x3):
    spec = pl.BlockSpec((1, H, W), lambda i: (0, 0, 0))
    return pl.pallas_call(
        _grad_index_body,
        grid=(NUM_IMGS,),
        in_specs=[spec, spec, spec, spec],
        out_specs=pl.BlockSpec((8, PER_TILE), lambda i: (i, 0)),
        out_shape=jax.ShapeDtypeStruct((NTILES, PER_TILE), jnp.int32),
    )(x0, x1, x2, x3)


@functools.cache
def _sc_hist_fn():
    mesh = plsc.VectorSubcoreMesh(core_axis_name="c", subcore_axis_name="s")

    @functools.partial(
        pl.kernel,
        mesh=mesh,
        out_type=jax.ShapeDtypeStruct((NTILES, HIST), jnp.float32),
        compiler_params=pltpu.CompilerParams(needs_layout_passes=False),
        scratch_types=[
            pltpu.VMEM((PER_TILE,), jnp.int32),
            pltpu.VMEM((HIST,), jnp.float32),
            pltpu.SemaphoreType.DMA,
        ],
    )
    def _sc_hist(idx_hbm, out_hbm, idx_v, hist_v, sem):
        c = lax.axis_index("c")
        s = lax.axis_index("s")
        wid = c * 16 + s
        cp = pltpu.async_copy(idx_hbm.at[wid], idx_v, sem)
        zero16 = jnp.zeros((16,), jnp.float32)

        def zbody(j, carry):
            for b in range(16):
                hist_v[pl.ds(j * 256 + b * 16, 16)] = zero16
            return carry

        lax.fori_loop(0, HIST // 256, zbody, 0)
        cp.wait()
        ones16 = jnp.ones((16,), jnp.float32)
        unroll = 16

        def body(j, carry):
            for b in range(unroll):
                iv = idx_v[pl.ds(j * (16 * unroll) + b * 16, 16)]
                plsc.addupdate_scatter(hist_v, [iv], ones16)
            return carry

        lax.fori_loop(0, PER_TILE // (16 * unroll), body, 0)
        pltpu.sync_copy(hist_v, out_hbm.at[wid])

    return _sc_hist


def _features_body(h_ref, o_ref):
    def img_hist(img):
        part = lax.slice(h_ref[...], (img * 8, 0), (img * 8 + 8, HIST))
        return jnp.reshape(jnp.sum(part, axis=0), (NUM_BINS, NUM_BINS))
    iif = lax.broadcasted_iota(jnp.int32, (NUM_BINS, NUM_BINS), 0).astype(jnp.float32)
    jjf = lax.broadcasted_iota(jnp.int32, (NUM_BINS, NUM_BINS), 1).astype(jnp.float32)
    corner = jnp.where((iif == 0.0) & (jjf == 0.0), float(H), 0.0)

    arf = lax.broadcasted_iota(jnp.int32, (1, NUM_BINS), 1).astype(jnp.float32)

    def feats(hc):
        g = hc + hc.T
        g = g / jnp.sum(g)
        energy = jnp.sum(g * g)
        rs = jnp.sum(g, axis=1).reshape(1, NUM_BINS)
        cs = jnp.sum(g, axis=0).reshape(1, NUM_BINS)
        mean_i = jnp.sum(arf * rs)
        mean_j = jnp.sum(arf * cs)
        std_i = jnp.sqrt(jnp.sum((mean_i - arf) ** 2 * cs))
        std_j = jnp.sqrt(jnp.sum((mean_j - arf) ** 2 * cs))
        b = (mean_j - arf) / (std_i * std_j)
        t = jnp.sum(g * b, axis=1).reshape(1, NUM_BINS)
        corr = jnp.sum((mean_i - arf) * t)
        corr = jnp.where((std_i == 0.0) | (std_j == 0.0), 0.0, corr)
        entropy = -jnp.sum(g * jnp.log(g + 1e-08))
        return energy, corr, entropy

    f = [feats(img_hist(i) - corner) for i in range(NUM_IMGS)]
    total = (jnp.abs(f[0][0] - f[1][0]) + jnp.abs(f[0][1] - f[1][1])
             + jnp.abs(f[0][2] - f[1][2]) + jnp.abs(f[2][0] - f[3][0])
             + jnp.abs(f[2][1] - f[3][1]) + jnp.abs(f[2][2] - f[3][2]))
    o_ref[0, 0] = total


def _features(hists):
    return pl.pallas_call(
        _features_body,
        out_specs=pl.BlockSpec(memory_space=pltpu.SMEM),
        out_shape=jax.ShapeDtypeStruct((1, 1), jnp.float32),
    )(hists)


def kernel(real_image, synthesized_image, real_outline, synthesized_outline):
    idx = _grad_index(real_image, synthesized_image, real_outline,
                      synthesized_outline)
    hist = _sc_hist_fn()(idx)
    total = _features(hist)
    return total.reshape(())


# speedup vs baseline: 1.2592x; 1.0936x over previous
import functools

import jax
import jax.numpy as jnp
import numpy as np
from jax import lax
from jax.experimental import pallas as pl
from jax.experimental.pallas import tpu as pltpu
from jax.experimental.pallas import tpu_sc as plsc

NUM_BINS = 256
H = 512
W = 512
NUM_IMGS = 4
HIST = NUM_BINS * NUM_BINS
TOTAL_BINS = NUM_IMGS * HIST
NTILES = 32
PER_TILE = (NUM_IMGS * H * W) // NTILES
CHUNK = 128
NCHUNK = PER_TILE // CHUNK
CORE_BINS = 2 * HIST
ZCHUNK = CORE_BINS // 16


def _grad_index_body(x0_ref, x1_ref, x2_ref, x3_ref, o_ref):
    img = pl.program_id(0)

    def one(x_ref):
        x = x_ref[0].astype(jnp.bfloat16).astype(jnp.float32)
        xd = jnp.pad(lax.slice(x, (0, 0), (H - 1, W)), ((1, 0), (0, 0)))
        xu = jnp.pad(lax.slice(x, (1, 0), (H, W)), ((0, 1), (0, 0)))
        vsm = (xd + 2.0 * x) + xu
        vdf = xu - xd

        def left(a):
            return jnp.pad(lax.slice(a, (0, 1), (H, W)), ((0, 0), (0, 1)))

        def right(a):
            return jnp.pad(lax.slice(a, (0, 0), (H, W - 1)), ((0, 0), (1, 0)))

        gx = left(vsm) - right(vsm)
        gy = (right(vdf) + 2.0 * vdf) + left(vdf)
        g = jnp.sqrt(gx * gx + gy * gy)
        gi = jnp.mod(jnp.floor(g * 255.0).astype(jnp.int32), NUM_BINS)
        q = jnp.pad(lax.slice(gi, (0, 1), (H, W)), ((0, 0), (0, 1)))
        col = lax.broadcasted_iota(jnp.int32, (H, W), 1)
        k = jnp.where(col < W - 1, gi * NUM_BINS + q, 0)
        o_ref[...] = jnp.reshape(k, (8, W * H // (8 * CHUNK), CHUNK))

    for n, r in enumerate((x0_ref, x1_ref, x2_ref, x3_ref)):
        @pl.when(img == n)
        def _(r=r):
            one(r)


def _grad_index(x0, x1, x2, x3):
    spec = pl.BlockSpec((1, H, W), lambda i: (0, 0, 0))
    return pl.pallas_call(
        _grad_index_body,
        grid=(NUM_IMGS,),
        in_specs=[spec, spec, spec, spec],
        out_specs=pl.BlockSpec((8, NCHUNK, CHUNK), lambda i: (i, 0, 0)),
        out_shape=jax.ShapeDtypeStruct((NTILES, NCHUNK, CHUNK), jnp.int32),
    )(x0, x1, x2, x3)


@functools.cache
def _sc_hist_fn():
    mesh = plsc.VectorSubcoreMesh(core_axis_name="c", subcore_axis_name="s")

    @functools.partial(
        pl.kernel,
        mesh=mesh,
        out_type=jax.ShapeDtypeStruct((2, 16, ZCHUNK), jnp.float32),
        scratch_types=[
            pltpu.VMEM((NCHUNK, CHUNK), jnp.int32),
            pltpu.VMEM((CHUNK,), jnp.float32),
            pltpu.VMEM_SHARED((CORE_BINS,), jnp.float32),
            pltpu.SemaphoreType.DMA,
        ],
    )
    def _sc_hist(idx_hbm, zeros_hbm, out_hbm, idx_v, ones_v, hist_sh, sem):
        c = lax.axis_index("c")
        s = lax.axis_index("s")
        wid = c * 16 + s
        pltpu.sync_copy(zeros_hbm, hist_sh.at[pl.ds(s * ZCHUNK, ZCHUNK)])
        pltpu.sync_copy(idx_hbm.at[wid], idx_v)
        for i in range(CHUNK // 16):
            ones_v[pl.ds(i * 16, 16)] = jnp.ones((16,), jnp.float32)
        plsc.subcore_barrier()

        grp = 16
        half = hist_sh.at[pl.ds(jnp.where(s < 8, 0, HIST), HIST)]

        def body(j, carry):
            cps = [
                pltpu.async_copy(ones_v, half.at[idx_v.at[j * grp + b]],
                                 sem, add=True)
                for b in range(grp)
            ]
            for cp in cps:
                cp.wait()
            return carry

        lax.fori_loop(0, NCHUNK // grp, body, 0)
        plsc.subcore_barrier()
        pltpu.sync_copy(hist_sh.at[pl.ds(s * ZCHUNK, ZCHUNK)], out_hbm.at[c, s])

    return _sc_hist


def _features_body(h_ref, o_ref):
    hsum = jnp.reshape(h_ref[...], (NUM_IMGS, NUM_BINS, NUM_BINS))
    iif = lax.broadcasted_iota(jnp.int32, (NUM_BINS, NUM_BINS), 0).astype(jnp.float32)
    jjf = lax.broadcasted_iota(jnp.int32, (NUM_BINS, NUM_BINS), 1).astype(jnp.float32)
    corner = jnp.where((iif == 0.0) & (jjf == 0.0), float(H), 0.0)

    arf = lax.broadcasted_iota(jnp.int32, (1, NUM_BINS), 1).astype(jnp.float32)

    def feats(hc):
        g = hc + hc.T
        g = g / jnp.sum(g)
        energy = jnp.sum(g * g)
        rs = jnp.sum(g, axis=1).reshape(1, NUM_BINS)
        cs = jnp.sum(g, axis=0).reshape(1, NUM_BINS)
        mean_i = jnp.sum(arf * rs)
        mean_j = jnp.sum(arf * cs)
        std_i = jnp.sqrt(jnp.sum((mean_i - arf) ** 2 * cs))
        std_j = jnp.sqrt(jnp.sum((mean_j - arf) ** 2 * cs))
        b = (mean_j - arf) / (std_i * std_j)
        t = jnp.sum(g * b, axis=1).reshape(1, NUM_BINS)
        corr = jnp.sum((mean_i - arf) * t)
        corr = jnp.where((std_i == 0.0) | (std_j == 0.0), 0.0, corr)
        entropy = -jnp.sum(g * jnp.log(g + 1e-08))
        return energy, corr, entropy

    f = [feats(hsum[i] - corner) for i in range(NUM_IMGS)]
    total = (jnp.abs(f[0][0] - f[1][0]) + jnp.abs(f[0][1] - f[1][1])
             + jnp.abs(f[0][2] - f[1][2]) + jnp.abs(f[2][0] - f[3][0])
             + jnp.abs(f[2][1] - f[3][1]) + jnp.abs(f[2][2] - f[3][2]))
    o_ref[0, 0] = total


def _features(hists):
    return pl.pallas_call(
        _features_body,
        out_specs=pl.BlockSpec(memory_space=pltpu.SMEM),
        out_shape=jax.ShapeDtypeStruct((1, 1), jnp.float32),
    )(hists)


def kernel(real_image, synthesized_image, real_outline, synthesized_outline):
    idx = _grad_index(real_image, synthesized_image, real_outline,
                      synthesized_outline)
    zeros = np.zeros((ZCHUNK,), np.float32)
    hist = _sc_hist_fn()(idx, zeros)
    total = _features(hist)
    return total.reshape(())
